# baseline (speedup 1.0000x reference)
import functools

import jax
import jax.numpy as jnp
from jax import lax
from jax.experimental import pallas as pl
from jax.experimental.pallas import tpu as pltpu
from jax.experimental.pallas import tpu_sc as plsc

N = 10000
E = 320000
F_IN = 128
HID = 16
C = 10

NC = 2
NS = 16
NW = NC * NS
CHUNK = 128
TPC = 80
EPAD = NW * TPC * CHUNK
ROWS_PER_TILE = 640
NPAD = NS * ROWS_PER_TILE
LAST = N - (NS - 1) * ROWS_PER_TILE
DEPTH = 8
LAG = 4

PACK = 128 // HID
NF = NPAD // PACK
BLKF = 128
GRID = NF // BLKF
BLKN = BLKF * PACK

_mesh = plsc.VectorSubcoreMesh(core_axis_name="c", subcore_axis_name="s")
_sc_params = pltpu.CompilerParams(use_tc_tiling_on_sc=False,
                                  needs_layout_passes=False)


T_PAD = PACK * (N % NF) + N // NF


def _slot(n):
    a = ((n >> 8) * 52429) >> 18
    r = n - a * NF
    return PACK * r + a


def _prep_body(e_ref, src_ref, dst_ref):
    src_ref[pl.ds(0, E)] = _slot(e_ref[0, :]).astype(jnp.int16)
    src_ref[pl.ds(E, EPAD - E)] = jnp.full((EPAD - E,), T_PAD, jnp.int16)
    dst_ref[pl.ds(0, E)] = _slot(e_ref[1, :]).astype(jnp.int16)
    dst_ref[pl.ds(E, EPAD - E)] = jnp.full((EPAD - E,), T_PAD, jnp.int16)


_prep = pl.pallas_call(
    _prep_body,
    out_shape=(
        jax.ShapeDtypeStruct((EPAD,), jnp.int16),
        jax.ShapeDtypeStruct((EPAD,), jnp.int16),
    ),
)

TPCC = TPC * CHUNK
NB = TPC // DEPTH


def _cvt_chunk(raw_ref, j, out2d_ref, b):
    for k in range(CHUNK // 32):
        v = raw_ref[pl.ds(j * CHUNK + 32 * k, 32)]
        lo, hi = plsc.unpack(v, format=plsc.PackFormat.INTERLEAVED)
        out2d_ref[b, pl.ds(32 * k, 16)] = lo
        out2d_ref[b, pl.ds(32 * k + 16, 16)] = hi


@functools.partial(
    pl.kernel,
    out_type=jax.ShapeDtypeStruct((NC * NPAD, HID), jnp.float32),
    mesh=_mesh,
    scratch_types=[
        pltpu.VMEM_SHARED((NPAD,), jnp.float32),
        pltpu.VMEM((TPCC,), jnp.int16),
        pltpu.VMEM((DEPTH, CHUNK), jnp.int32),
        pltpu.VMEM((CHUNK,), jnp.float32),
        pltpu.VMEM((ROWS_PER_TILE + 16,), jnp.float32),
        pltpu.VMEM((ROWS_PER_TILE, HID), jnp.float32),
        pltpu.SemaphoreType.DMA,
    ],
    compiler_params=_sc_params,
)
def _deg_kernel(dst_hbm, out_hbm, deg_sh, raw_v, idx_v, ones_v, bounce_v,
                rep_v, sem):
    c = lax.axis_index("c")
    s = lax.axis_index("s")
    wid = s * NC + c

    idx_load = pltpu.async_copy(dst_hbm.at[pl.ds(wid * TPCC, TPCC)], raw_v, sem)

    for i in range(CHUNK // 16):
        ones_v[pl.ds(i * 16, 16)] = jnp.ones((16,), jnp.float32)

    def zero_body(i, _):
        bounce_v[pl.ds(i * 16, 16)] = jnp.zeros((16,), jnp.float32)
        return 0

    lax.fori_loop(0, ROWS_PER_TILE // 16 + 1, zero_body, 0)
    pltpu.sync_copy(bounce_v.at[pl.ds(0, ROWS_PER_TILE)],
                    deg_sh.at[pl.ds(s * ROWS_PER_TILE, ROWS_PER_TILE)])
    idx_load.wait()
    plsc.subcore_barrier()

    def scat(b):
        return pltpu.make_async_copy(ones_v, deg_sh.at[idx_v.at[b]], sem)

    def body(g, _):
        for b in range(DEPTH):
            @pl.when(g > 0)
            def _():
                scat(b).wait()

            _cvt_chunk(raw_v, g * DEPTH + b, idx_v, b)
            scat(b).start(add=True)
        return 0

    lax.fori_loop(0, NB, body, 0)
    for b in range(DEPTH):
        scat(b).wait()
    plsc.subcore_barrier()

    pltpu.sync_copy(deg_sh.at[pl.ds(s * ROWS_PER_TILE, ROWS_PER_TILE)],
                    bounce_v.at[pl.ds(0, ROWS_PER_TILE)])

    def rep_body(i, _):
        v = bounce_v[pl.ds(i, 16)]
        rep_v[i, :] = jnp.full((HID,), v[0], jnp.float32)
        return 0

    lax.fori_loop(0, ROWS_PER_TILE, rep_body, 0)

    pltpu.sync_copy(
        rep_v, out_hbm.at[pl.ds(c * NPAD + s * ROWS_PER_TILE, ROWS_PER_TILE)]
    )


@functools.partial(
    pl.kernel,
    out_type=jax.ShapeDtypeStruct((NC * NPAD, HID), jnp.float32),
    mesh=_mesh,
    scratch_types=[
        pltpu.VMEM_SHARED((NPAD, HID), jnp.float32),
        pltpu.VMEM_SHARED((NPAD, HID), jnp.float32),
        pltpu.VMEM((TPCC,), jnp.int16),
        pltpu.VMEM((TPCC,), jnp.int16),
        pltpu.VMEM((DEPTH, CHUNK), jnp.int32),
        pltpu.VMEM((DEPTH, CHUNK), jnp.int32),
        pltpu.VMEM((DEPTH, CHUNK, HID), jnp.float32),
        pltpu.VMEM((ROWS_PER_TILE, HID), jnp.float32),
        pltpu.SemaphoreType.DMA,
        pltpu.SemaphoreType.DMA,
    ],
    compiler_params=_sc_params,
)
def _prop_kernel(table_hbm, src_hbm, dst_hbm, out_hbm,
                 agg_sh, table_sh, sraw_v, draw_v, isrc_v, idst_v, rows_v,
                 bounce_v, sem_g, sem_s):
    c = lax.axis_index("c")
    s = lax.axis_index("s")
    wid = s * NC + c

    src_load = pltpu.async_copy(src_hbm.at[pl.ds(wid * TPCC, TPCC)], sraw_v,
                                sem_g)
    dst_load = pltpu.async_copy(dst_hbm.at[pl.ds(wid * TPCC, TPCC)], draw_v,
                                sem_g)

    pltpu.sync_copy(
        table_hbm.at[pl.ds(s * ROWS_PER_TILE, ROWS_PER_TILE)],
        table_sh.at[pl.ds(s * ROWS_PER_TILE, ROWS_PER_TILE)])

    def zero_body(i, _):
        bounce_v[i, :] = jnp.zeros((HID,), jnp.float32)
        return 0

    lax.fori_loop(0, ROWS_PER_TILE, zero_body, 0)
    pltpu.sync_copy(bounce_v, agg_sh.at[pl.ds(s * ROWS_PER_TILE, ROWS_PER_TILE)])
    src_load.wait()
    dst_load.wait()
    plsc.subcore_barrier()

    def gath(b):
        return pltpu.make_async_copy(table_sh.at[isrc_v.at[b]], rows_v.at[b],
                                     sem_g)

    def scat(b):
        return pltpu.make_async_copy(rows_v.at[b], agg_sh.at[idst_v.at[b]],
                                     sem_s)

    def body(g, _):
        for b in range(DEPTH):
            @pl.when(g > 0)
            def _():
                scat(b).wait()

            j = g * DEPTH + b
            _cvt_chunk(sraw_v, j, isrc_v, b)
            _cvt_chunk(draw_v, j, idst_v, b)
            gath(b).start()
        for b in range(DEPTH):
            gath(b).wait()
            scat(b).start(add=True)
        return 0

    lax.fori_loop(0, NB, body, 0)
    for b in range(DEPTH):
        scat(b).wait()
    plsc.subcore_barrier()

    pltpu.sync_copy(agg_sh.at[pl.ds(s * ROWS_PER_TILE, ROWS_PER_TILE)], bounce_v)

    pltpu.sync_copy(
        bounce_v, out_hbm.at[pl.ds(c * NPAD + s * ROWS_PER_TILE, ROWS_PER_TILE)]
    )


def _tc1_body(d0_ref, d1_ref, *rest):
    xrefs = rest[:PACK]
    w1s_ref, scaled_ref, dinv_ref = rest[PACK], rest[PACK + 1], rest[PACK + 2]
    dinv = lax.rsqrt(d0_ref[...] + d1_ref[...] + 1.0)
    xcat = jnp.concatenate([xr[0] for xr in xrefs], axis=1)
    hf = jnp.dot(xcat, w1s_ref[...], preferred_element_type=jnp.float32)
    scaled_ref[...] = hf * dinv
    dinv_ref[...] = dinv


def _xa_spec(a):
    return pl.BlockSpec((1, BLKF, F_IN), lambda g, a=a: (a, g, 0))


_tc1 = pl.pallas_call(
    _tc1_body,
    grid=(GRID,),
    in_specs=[
        pl.BlockSpec((BLKF, 128), lambda g: (g, 0)),
        pl.BlockSpec((BLKF, 128), lambda g: (g + GRID, 0)),
    ] + [_xa_spec(a) for a in range(PACK)] + [
        pl.BlockSpec((PACK * F_IN, 128), lambda g: (0, 0)),
    ],
    out_specs=(
        pl.BlockSpec((BLKF, 128), lambda g: (g, 0)),
        pl.BlockSpec((BLKF, 128), lambda g: (g, 0)),
    ),
    out_shape=(
        jax.ShapeDtypeStruct((NF, 128), jnp.float32),
        jax.ShapeDtypeStruct((NF, 128), jnp.float32),
    ),
)


def _tc2_body(a0_ref, a1_ref, scaled1_ref, dinv_ref, b1_ref, w2blk_ref,
              out_ref):
    dinv = dinv_ref[...]
    h1 = jnp.maximum(
        dinv * (a0_ref[...] + a1_ref[...] + scaled1_ref[...]) + b1_ref[...],
        0.0)
    h2 = jnp.dot(h1, w2blk_ref[...], preferred_element_type=jnp.float32)
    out_ref[...] = h2 * dinv


_tc2 = pl.pallas_call(
    _tc2_body,
    grid=(GRID,),
    in_specs=[
        pl.BlockSpec((BLKF, 128), lambda g: (g, 0)),
        pl.BlockSpec((BLKF, 128), lambda g: (g + GRID, 0)),
        pl.BlockSpec((BLKF, 128), lambda g: (g, 0)),
        pl.BlockSpec((BLKF, 128), lambda g: (g, 0)),
        pl.BlockSpec((1, 128), lambda g: (0, 0)),
        pl.BlockSpec((128, 128), lambda g: (0, 0)),
    ],
    out_specs=pl.BlockSpec((BLKF, 128), lambda g: (g, 0)),
    out_shape=jax.ShapeDtypeStruct((NF, 128), jnp.float32),
)


def _tc3_body(a0_ref, a1_ref, scaled2_ref, dinv_ref, b2_ref, g8_ref, out_ref):
    logits = dinv_ref[...] * (
        a0_ref[...] + a1_ref[...] + scaled2_ref[...]
    ) + b2_ref[...]
    m = jnp.max(logits, axis=1, keepdims=True)
    e = jnp.exp(logits - m)
    s = jnp.dot(e, g8_ref[...], preferred_element_type=jnp.float32)
    out_ref[...] = logits - m - jnp.log(s)


_tc3 = pl.pallas_call(
    _tc3_body,
    grid=(GRID,),
    in_specs=[
        pl.BlockSpec((BLKF, 128), lambda g: (g, 0)),
        pl.BlockSpec((BLKF, 128), lambda g: (g + GRID, 0)),
        pl.BlockSpec((BLKF, 128), lambda g: (g, 0)),
        pl.BlockSpec((BLKF, 128), lambda g: (g, 0)),
        pl.BlockSpec((1, 128), lambda g: (0, 0)),
        pl.BlockSpec((128, 128), lambda g: (0, 0)),
    ],
    out_specs=pl.BlockSpec((BLKF, 128), lambda g: (g, 0)),
    out_shape=jax.ShapeDtypeStruct((NF, 128), jnp.float32),
)


def kernel(x, edge_index, W1, b1, W2, b2):
    srcp, dstp = _prep(edge_index.astype(jnp.int32))

    w2p = jnp.zeros((HID, HID), jnp.float32).at[:, :C].set(W2)
    w2blk = jnp.kron(jnp.eye(PACK, dtype=jnp.float32), w2p)
    g8 = jnp.kron(jnp.eye(PACK, dtype=jnp.float32),
                  jnp.ones((HID, HID), jnp.float32))
    b1f = jnp.tile(b1.reshape(1, HID), (1, PACK))
    b2p = jnp.full((HID,), -1e30, jnp.float32).at[:C].set(b2)
    b2f = jnp.tile(b2p.reshape(1, HID), (1, PACK))

    xp = jnp.concatenate(
        [x, jnp.zeros((NPAD - N, F_IN), jnp.float32)]).reshape(PACK, NF, F_IN)
    w1s = jnp.kron(jnp.eye(PACK, dtype=jnp.float32), W1)

    degrep = _deg_kernel(dstp)
    degf = degrep.reshape(NC * NF, 128)

    scaled1, dinv = _tc1(degf, degf, *([xp] * PACK), w1s)

    agg1 = _prop_kernel(scaled1.reshape(NPAD, HID), srcp, dstp)
    agg1f = agg1.reshape(NC * NF, 128)
    scaled2 = _tc2(agg1f, agg1f, scaled1, dinv, b1f, w2blk)

    agg2 = _prop_kernel(scaled2.reshape(NPAD, HID), srcp, dstp)
    agg2f = agg2.reshape(NC * NF, 128)
    lp = _tc3(agg2f, agg2f, scaled2, dinv, b2f, g8)

    return lp.reshape(NF, PACK, HID).transpose(1, 0, 2).reshape(NPAD, HID)[:N, :C]


# speedup vs baseline: 1.1337x; 1.1337x over previous
import functools

import jax
import jax.numpy as jnp
from jax import lax
from jax.experimental import pallas as pl
from jax.experimental.pallas import tpu as pltpu
from jax.experimental.pallas import tpu_sc as plsc

N = 10000
E = 320000
F_IN = 128
HID = 16
C = 10

NC = 2
NS = 16
NW = NC * NS
CHUNK = 128
TPC = 80
EPAD = NW * TPC * CHUNK
ROWS_PER_TILE = 640
NPAD = NS * ROWS_PER_TILE
LAST = N - (NS - 1) * ROWS_PER_TILE
DEPTH = 8
LAG = 4

PACK = 128 // HID
NF = NPAD // PACK
BLKF = 256
GRID = NF // BLKF
BLKN = BLKF * PACK

_mesh = plsc.VectorSubcoreMesh(core_axis_name="c", subcore_axis_name="s")
_sc_params = pltpu.CompilerParams(use_tc_tiling_on_sc=False)


T_PAD = PACK * (N % NF) + N // NF


def _slot(n):
    a = ((n >> 8) * 52429) >> 18
    r = n - a * NF
    return PACK * r + a


def _prep_body(e_ref, src_ref, dst_ref):
    src_ref[pl.ds(0, E)] = _slot(e_ref[0, :])
    src_ref[pl.ds(E, EPAD - E)] = jnp.full((EPAD - E,), T_PAD, jnp.int32)
    dst_ref[pl.ds(0, E)] = _slot(e_ref[1, :])
    dst_ref[pl.ds(E, EPAD - E)] = jnp.full((EPAD - E,), T_PAD, jnp.int32)


_prep = pl.pallas_call(
    _prep_body,
    out_shape=(
        jax.ShapeDtypeStruct((EPAD,), jnp.int32),
        jax.ShapeDtypeStruct((EPAD,), jnp.int32),
    ),
)


@functools.partial(
    pl.kernel,
    out_type=jax.ShapeDtypeStruct((NC * NPAD, HID), jnp.float32),
    mesh=_mesh,
    scratch_types=[
        pltpu.VMEM_SHARED((NPAD,), jnp.float32),
        pltpu.VMEM((TPC, CHUNK), jnp.int32),
        pltpu.VMEM((CHUNK,), jnp.float32),
        pltpu.VMEM((ROWS_PER_TILE + 16,), jnp.float32),
        pltpu.VMEM((ROWS_PER_TILE, HID), jnp.float32),
        pltpu.SemaphoreType.DMA,
    ],
    compiler_params=_sc_params,
)
def _deg_kernel(dst_hbm, out_hbm, deg_sh, idx_v, ones_v, bounce_v, rep_v, sem):
    c = lax.axis_index("c")
    s = lax.axis_index("s")
    wid = s * NC + c

    idx_load = pltpu.async_copy(dst_hbm.at[wid], idx_v, sem)

    for i in range(CHUNK // 16):
        ones_v[pl.ds(i * 16, 16)] = jnp.ones((16,), jnp.float32)

    def zero_body(i, _):
        bounce_v[pl.ds(i * 16, 16)] = jnp.zeros((16,), jnp.float32)
        return 0

    lax.fori_loop(0, ROWS_PER_TILE // 16 + 1, zero_body, 0)
    pltpu.sync_copy(bounce_v.at[pl.ds(0, ROWS_PER_TILE)],
                    deg_sh.at[pl.ds(s * ROWS_PER_TILE, ROWS_PER_TILE)])
    idx_load.wait()
    plsc.subcore_barrier()

    def scat(j):
        return pltpu.make_async_copy(ones_v, deg_sh.at[idx_v.at[j]], sem)

    def body(j, _):
        scat(j).start(add=True)

        @pl.when(j >= DEPTH)
        def _():
            scat(0).wait()

        return 0

    lax.fori_loop(0, TPC, body, 0)
    for _ in range(DEPTH):
        scat(0).wait()
    plsc.subcore_barrier()

    pltpu.sync_copy(deg_sh.at[pl.ds(s * ROWS_PER_TILE, ROWS_PER_TILE)],
                    bounce_v.at[pl.ds(0, ROWS_PER_TILE)])

    def rep_body(i, _):
        v = bounce_v[pl.ds(i, 16)]
        rep_v[i, :] = jnp.full((HID,), v[0], jnp.float32)
        return 0

    lax.fori_loop(0, ROWS_PER_TILE, rep_body, 0)

    pltpu.sync_copy(
        rep_v, out_hbm.at[pl.ds(c * NPAD + s * ROWS_PER_TILE, ROWS_PER_TILE)]
    )


@functools.partial(
    pl.kernel,
    out_type=jax.ShapeDtypeStruct((NC * NPAD, HID), jnp.float32),
    mesh=_mesh,
    scratch_types=[
        pltpu.VMEM_SHARED((NPAD, HID), jnp.float32),
        pltpu.VMEM_SHARED((NPAD, HID), jnp.float32),
        pltpu.VMEM((TPC, CHUNK), jnp.int32),
        pltpu.VMEM((TPC, CHUNK), jnp.int32),
        pltpu.VMEM((DEPTH, CHUNK, HID), jnp.float32),
        pltpu.VMEM((ROWS_PER_TILE, HID), jnp.float32),
        pltpu.SemaphoreType.DMA,
        pltpu.SemaphoreType.DMA,
    ],
    compiler_params=_sc_params,
)
def _prop_kernel(table_hbm, src_hbm, dst_hbm, out_hbm,
                 agg_sh, table_sh, isrc_v, idst_v, rows_v, bounce_v,
                 sem_g, sem_s):
    c = lax.axis_index("c")
    s = lax.axis_index("s")
    wid = s * NC + c

    src_load = pltpu.async_copy(src_hbm.at[wid], isrc_v, sem_g)
    dst_load = pltpu.async_copy(dst_hbm.at[wid], idst_v, sem_g)

    pltpu.sync_copy(
        table_hbm.at[pl.ds(s * ROWS_PER_TILE, ROWS_PER_TILE)],
        table_sh.at[pl.ds(s * ROWS_PER_TILE, ROWS_PER_TILE)])

    def zero_body(i, _):
        bounce_v[i, :] = jnp.zeros((HID,), jnp.float32)
        return 0

    lax.fori_loop(0, ROWS_PER_TILE, zero_body, 0)
    pltpu.sync_copy(bounce_v, agg_sh.at[pl.ds(s * ROWS_PER_TILE, ROWS_PER_TILE)])
    src_load.wait()
    dst_load.wait()
    plsc.subcore_barrier()

    def gath(j, b):
        return pltpu.make_async_copy(table_sh.at[isrc_v.at[j]], rows_v.at[b],
                                     sem_g)

    def scat(j, b):
        return pltpu.make_async_copy(rows_v.at[b], agg_sh.at[idst_v.at[j]],
                                     sem_s)

    def body(j, _):
        b = lax.rem(j, DEPTH)

        @pl.when(j >= DEPTH)
        def _():
            scat(0, 0).wait()

        gath(j, b).start()

        @pl.when(j >= LAG)
        def _():
            gath(0, 0).wait()
            jj = j - LAG
            scat(jj, lax.rem(jj, DEPTH)).start(add=True)

        return 0

    lax.fori_loop(0, TPC, body, 0)

    def tail(j, _):
        gath(0, 0).wait()
        jj = j - LAG
        scat(jj, lax.rem(jj, DEPTH)).start(add=True)
        return 0

    lax.fori_loop(TPC, TPC + LAG, tail, 0)
    for _ in range(DEPTH):
        scat(0, 0).wait()
    plsc.subcore_barrier()

    pltpu.sync_copy(agg_sh.at[pl.ds(s * ROWS_PER_TILE, ROWS_PER_TILE)], bounce_v)

    pltpu.sync_copy(
        bounce_v, out_hbm.at[pl.ds(c * NPAD + s * ROWS_PER_TILE, ROWS_PER_TILE)]
    )


def _tc1_body(d0_ref, d1_ref, *rest):
    xrefs = rest[:PACK]
    w1s_ref, scaled_ref, dinv_ref = rest[PACK], rest[PACK + 1], rest[PACK + 2]
    dinv = lax.rsqrt(d0_ref[...] + d1_ref[...] + 1.0)
    xcat = jnp.concatenate([xr[...] for xr in xrefs], axis=1)
    ri = lax.broadcasted_iota(jnp.int32, (BLKF, PACK * F_IN), 0) \
        + pl.program_id(0) * BLKF
    ci = lax.broadcasted_iota(jnp.int32, (BLKF, PACK * F_IN), 1)
    pad = (ci >= (PACK - 1) * F_IN) & (ri >= (N % NF))
    xcat = jnp.where(pad, 0.0, xcat)
    hf = jnp.dot(xcat, w1s_ref[...], preferred_element_type=jnp.float32)
    scaled_ref[...] = hf * dinv
    dinv_ref[...] = dinv


def _xa_spec(a):
    return pl.BlockSpec((BLKF, F_IN), lambda g, a=a: (g + a * (NF // BLKF), 0))


_tc1 = pl.pallas_call(
    _tc1_body,
    grid=(GRID,),
    in_specs=[
        pl.BlockSpec((BLKF, 128), lambda g: (g, 0)),
        pl.BlockSpec((BLKF, 128), lambda g: (g + GRID, 0)),
    ] + [_xa_spec(a) for a in range(PACK)] + [
        pl.BlockSpec((PACK * F_IN, 128), lambda g: (0, 0)),
    ],
    out_specs=(
        pl.BlockSpec((BLKF, 128), lambda g: (g, 0)),
        pl.BlockSpec((BLKF, 128), lambda g: (g, 0)),
    ),
    out_shape=(
        jax.ShapeDtypeStruct((NF, 128), jnp.float32),
        jax.ShapeDtypeStruct((NF, 128), jnp.float32),
    ),
)


def _tc2_body(a0_ref, a1_ref, scaled1_ref, dinv_ref, b1_ref, w2blk_ref,
              out_ref):
    dinv = dinv_ref[...]
    h1 = jnp.maximum(
        dinv * (a0_ref[...] + a1_ref[...] + scaled1_ref[...]) + b1_ref[...],
        0.0)
    h2 = jnp.dot(h1, w2blk_ref[...], preferred_element_type=jnp.float32)
    out_ref[...] = h2 * dinv


_tc2 = pl.pallas_call(
    _tc2_body,
    grid=(GRID,),
    in_specs=[
        pl.BlockSpec((BLKF, 128), lambda g: (g, 0)),
        pl.BlockSpec((BLKF, 128), lambda g: (g + GRID, 0)),
        pl.BlockSpec((BLKF, 128), lambda g: (g, 0)),
        pl.BlockSpec((BLKF, 128), lambda g: (g, 0)),
        pl.BlockSpec((1, 128), lambda g: (0, 0)),
        pl.BlockSpec((128, 128), lambda g: (0, 0)),
    ],
    out_specs=pl.BlockSpec((BLKF, 128), lambda g: (g, 0)),
    out_shape=jax.ShapeDtypeStruct((NF, 128), jnp.float32),
)


def _tc3_body(a0_ref, a1_ref, scaled2_ref, dinv_ref, b2_ref, g8_ref, out_ref):
    logits = dinv_ref[...] * (
        a0_ref[...] + a1_ref[...] + scaled2_ref[...]
    ) + b2_ref[...]
    m = jnp.max(logits, axis=1, keepdims=True)
    e = jnp.exp(logits - m)
    s = jnp.dot(e, g8_ref[...], preferred_element_type=jnp.float32)
    out_ref[...] = logits - m - jnp.log(s)


_tc3 = pl.pallas_call(
    _tc3_body,
    grid=(GRID,),
    in_specs=[
        pl.BlockSpec((BLKF, 128), lambda g: (g, 0)),
        pl.BlockSpec((BLKF, 128), lambda g: (g + GRID, 0)),
        pl.BlockSpec((BLKF, 128), lambda g: (g, 0)),
        pl.BlockSpec((BLKF, 128), lambda g: (g, 0)),
        pl.BlockSpec((1, 128), lambda g: (0, 0)),
        pl.BlockSpec((128, 128), lambda g: (0, 0)),
    ],
    out_specs=pl.BlockSpec((BLKF, 128), lambda g: (g, 0)),
    out_shape=jax.ShapeDtypeStruct((NF, 128), jnp.float32),
)


def kernel(x, edge_index, W1, b1, W2, b2):
    srcp, dstp = _prep(edge_index.astype(jnp.int32))
    src3d = srcp.reshape(NW, TPC, CHUNK)
    dst3d = dstp.reshape(NW, TPC, CHUNK)

    w2p = jnp.zeros((HID, HID), jnp.float32).at[:, :C].set(W2)
    w2blk = jnp.kron(jnp.eye(PACK, dtype=jnp.float32), w2p)
    g8 = jnp.kron(jnp.eye(PACK, dtype=jnp.float32),
                  jnp.ones((HID, HID), jnp.float32))
    b1f = jnp.tile(b1.reshape(1, HID), (1, PACK))
    b2p = jnp.full((HID,), -1e30, jnp.float32).at[:C].set(b2)
    b2f = jnp.tile(b2p.reshape(1, HID), (1, PACK))

    w1s = jnp.kron(jnp.eye(PACK, dtype=jnp.float32), W1)

    degrep = _deg_kernel(dst3d)
    degf = degrep.reshape(NC * NF, 128)

    scaled1, dinv = _tc1(degf, degf, *([x] * PACK), w1s)

    agg1 = _prop_kernel(scaled1.reshape(NPAD, HID), src3d, dst3d)
    agg1f = agg1.reshape(NC * NF, 128)
    scaled2 = _tc2(agg1f, agg1f, scaled1, dinv, b1f, w2blk)

    agg2 = _prop_kernel(scaled2.reshape(NPAD, HID), src3d, dst3d)
    agg2f = agg2.reshape(NC * NF, 128)
    lp = _tc3(agg2f, agg2f, scaled2, dinv, b2f, g8)

    return lp.reshape(NF, PACK, HID).transpose(1, 0, 2).reshape(NPAD, HID)[:N, :C]


# speedup vs baseline: 1.1550x; 1.0188x over previous
import functools

import jax
import jax.numpy as jnp
from jax import lax
from jax.experimental import pallas as pl
from jax.experimental.pallas import tpu as pltpu
from jax.experimental.pallas import tpu_sc as plsc

N = 10000
E = 320000
F_IN = 128
HID = 16
C = 10

NC = 2
NS = 16
NW = NC * NS
CHUNK = 128
TPC = 80
EPAD = NW * TPC * CHUNK
ROWS_PER_TILE = 640
NPAD = NS * ROWS_PER_TILE
LAST = N - (NS - 1) * ROWS_PER_TILE
DEPTH = 8
LAG = 4
NCHUNKS = EPAD // CHUNK
TPC_F = 96
TPC_S = 64

PACK = 128 // HID
NF = NPAD // PACK
BLKF = 256
GRID = NF // BLKF
BLKN = BLKF * PACK

_mesh = plsc.VectorSubcoreMesh(core_axis_name="c", subcore_axis_name="s")
_sc_params = pltpu.CompilerParams(use_tc_tiling_on_sc=False)


T_PAD = PACK * (N % NF) + N // NF


def _slot(n):
    a = ((n >> 8) * 52429) >> 18
    r = n - a * NF
    return PACK * r + a


def _prep_body(e_ref, src_ref, dst_ref):
    src_ref[pl.ds(0, E)] = _slot(e_ref[0, :])
    src_ref[pl.ds(E, EPAD - E)] = jnp.full((EPAD - E,), T_PAD, jnp.int32)
    dst_ref[pl.ds(0, E)] = _slot(e_ref[1, :])
    dst_ref[pl.ds(E, EPAD - E)] = jnp.full((EPAD - E,), T_PAD, jnp.int32)


_prep = pl.pallas_call(
    _prep_body,
    out_shape=(
        jax.ShapeDtypeStruct((EPAD,), jnp.int32),
        jax.ShapeDtypeStruct((EPAD,), jnp.int32),
    ),
)


@functools.partial(
    pl.kernel,
    out_type=jax.ShapeDtypeStruct((NC * NPAD, HID), jnp.float32),
    mesh=_mesh,
    scratch_types=[
        pltpu.VMEM_SHARED((NPAD,), jnp.float32),
        pltpu.VMEM((TPC_F, CHUNK), jnp.int32),
        pltpu.VMEM((CHUNK,), jnp.float32),
        pltpu.VMEM((ROWS_PER_TILE + 16,), jnp.float32),
        pltpu.VMEM((ROWS_PER_TILE, HID), jnp.float32),
        pltpu.SemaphoreType.DMA,
    ],
    compiler_params=_sc_params,
)
def _deg_kernel(dst_hbm, out_hbm, deg_sh, idx_v, ones_v, bounce_v, rep_v, sem):
    c = lax.axis_index("c")
    s = lax.axis_index("s")
    tpc_c = jnp.where(c == 0, TPC_F, TPC_S)

    for i in range(CHUNK // 16):
        ones_v[pl.ds(i * 16, 16)] = jnp.ones((16,), jnp.float32)

    def zero_body(i, _):
        bounce_v[pl.ds(i * 16, 16)] = jnp.zeros((16,), jnp.float32)
        return 0

    lax.fori_loop(0, ROWS_PER_TILE // 16 + 1, zero_body, 0)
    pltpu.sync_copy(bounce_v.at[pl.ds(0, ROWS_PER_TILE)],
                    deg_sh.at[pl.ds(s * ROWS_PER_TILE, ROWS_PER_TILE)])

    @pl.when(c == 0)
    def _():
        pltpu.sync_copy(dst_hbm.at[pl.ds(s * TPC_F, TPC_F)], idx_v)

    @pl.when(c == 1)
    def _():
        pltpu.sync_copy(dst_hbm.at[pl.ds(NS * TPC_F + s * TPC_S, TPC_S)],
                        idx_v.at[pl.ds(0, TPC_S)])

    plsc.subcore_barrier()

    def scat(j):
        return pltpu.make_async_copy(ones_v, deg_sh.at[idx_v.at[j]], sem)

    def body(j, _):
        scat(j).start(add=True)

        @pl.when(j >= DEPTH)
        def _():
            scat(0).wait()

        return 0

    lax.fori_loop(0, tpc_c, body, 0)
    for _ in range(DEPTH):
        scat(0).wait()
    plsc.subcore_barrier()

    pltpu.sync_copy(deg_sh.at[pl.ds(s * ROWS_PER_TILE, ROWS_PER_TILE)],
                    bounce_v.at[pl.ds(0, ROWS_PER_TILE)])

    def rep_body(i, _):
        v = bounce_v[pl.ds(i, 16)]
        rep_v[i, :] = jnp.full((HID,), v[0], jnp.float32)
        return 0

    lax.fori_loop(0, ROWS_PER_TILE, rep_body, 0)

    pltpu.sync_copy(
        rep_v, out_hbm.at[pl.ds(c * NPAD + s * ROWS_PER_TILE, ROWS_PER_TILE)]
    )


@functools.partial(
    pl.kernel,
    out_type=jax.ShapeDtypeStruct((NC * NPAD, HID), jnp.float32),
    mesh=_mesh,
    scratch_types=[
        pltpu.VMEM_SHARED((NPAD, HID), jnp.float32),
        pltpu.VMEM_SHARED((NPAD, HID), jnp.float32),
        pltpu.VMEM((TPC_F, CHUNK), jnp.int32),
        pltpu.VMEM((TPC_F, CHUNK), jnp.int32),
        pltpu.VMEM((DEPTH, CHUNK, HID), jnp.float32),
        pltpu.VMEM((ROWS_PER_TILE, HID), jnp.float32),
        pltpu.SemaphoreType.DMA,
        pltpu.SemaphoreType.DMA,
    ],
    compiler_params=_sc_params,
)
def _prop_kernel(table_hbm, src_hbm, dst_hbm, out_hbm,
                 agg_sh, table_sh, isrc_v, idst_v, rows_v, bounce_v,
                 sem_g, sem_s):
    c = lax.axis_index("c")
    s = lax.axis_index("s")
    tpc_c = jnp.where(c == 0, TPC_F, TPC_S)

    pltpu.sync_copy(
        table_hbm.at[pl.ds(s * ROWS_PER_TILE, ROWS_PER_TILE)],
        table_sh.at[pl.ds(s * ROWS_PER_TILE, ROWS_PER_TILE)])

    def zero_body(i, _):
        bounce_v[i, :] = jnp.zeros((HID,), jnp.float32)
        return 0

    lax.fori_loop(0, ROWS_PER_TILE, zero_body, 0)
    pltpu.sync_copy(bounce_v, agg_sh.at[pl.ds(s * ROWS_PER_TILE, ROWS_PER_TILE)])

    @pl.when(c == 0)
    def _():
        pltpu.sync_copy(src_hbm.at[pl.ds(s * TPC_F, TPC_F)], isrc_v)
        pltpu.sync_copy(dst_hbm.at[pl.ds(s * TPC_F, TPC_F)], idst_v)

    @pl.when(c == 1)
    def _():
        base = NS * TPC_F + s * TPC_S
        pltpu.sync_copy(src_hbm.at[pl.ds(base, TPC_S)],
                        isrc_v.at[pl.ds(0, TPC_S)])
        pltpu.sync_copy(dst_hbm.at[pl.ds(base, TPC_S)],
                        idst_v.at[pl.ds(0, TPC_S)])

    plsc.subcore_barrier()

    def gath(j, b):
        return pltpu.make_async_copy(table_sh.at[isrc_v.at[j]], rows_v.at[b],
                                     sem_g)

    def scat(j, b):
        return pltpu.make_async_copy(rows_v.at[b], agg_sh.at[idst_v.at[j]],
                                     sem_s)

    def body(j, _):
        b = lax.rem(j, DEPTH)

        @pl.when(j >= DEPTH)
        def _():
            scat(0, 0).wait()

        gath(j, b).start()

        @pl.when(j >= LAG)
        def _():
            gath(0, 0).wait()
            jj = j - LAG
            scat(jj, lax.rem(jj, DEPTH)).start(add=True)

        return 0

    lax.fori_loop(0, tpc_c, body, 0)

    def tail(j, _):
        gath(0, 0).wait()
        jj = j - LAG
        scat(jj, lax.rem(jj, DEPTH)).start(add=True)
        return 0

    lax.fori_loop(tpc_c, tpc_c + LAG, tail, 0)
    for _ in range(DEPTH):
        scat(0, 0).wait()
    plsc.subcore_barrier()

    pltpu.sync_copy(agg_sh.at[pl.ds(s * ROWS_PER_TILE, ROWS_PER_TILE)], bounce_v)

    pltpu.sync_copy(
        bounce_v, out_hbm.at[pl.ds(c * NPAD + s * ROWS_PER_TILE, ROWS_PER_TILE)]
    )


def _tc1_body(d0_ref, d1_ref, *rest):
    xrefs = rest[:PACK]
    w1s_ref, scaled_ref, dinv_ref = rest[PACK], rest[PACK + 1], rest[PACK + 2]
    dinv = lax.rsqrt(d0_ref[...] + d1_ref[...] + 1.0)
    xcat = jnp.concatenate([xr[...] for xr in xrefs], axis=1)
    ri = lax.broadcasted_iota(jnp.int32, (BLKF, PACK * F_IN), 0) \
        + pl.program_id(0) * BLKF
    ci = lax.broadcasted_iota(jnp.int32, (BLKF, PACK * F_IN), 1)
    pad = (ci >= (PACK - 1) * F_IN) & (ri >= (N % NF))
    xcat = jnp.where(pad, 0.0, xcat)
    hf = jnp.dot(xcat, w1s_ref[...], preferred_element_type=jnp.float32)
    scaled_ref[...] = hf * dinv
    dinv_ref[...] = dinv


def _xa_spec(a):
    return pl.BlockSpec((BLKF, F_IN), lambda g, a=a: (g + a * (NF // BLKF), 0))


_tc1 = pl.pallas_call(
    _tc1_body,
    grid=(GRID,),
    in_specs=[
        pl.BlockSpec((BLKF, 128), lambda g: (g, 0)),
        pl.BlockSpec((BLKF, 128), lambda g: (g + GRID, 0)),
    ] + [_xa_spec(a) for a in range(PACK)] + [
        pl.BlockSpec((PACK * F_IN, 128), lambda g: (0, 0)),
    ],
    out_specs=(
        pl.BlockSpec((BLKF, 128), lambda g: (g, 0)),
        pl.BlockSpec((BLKF, 128), lambda g: (g, 0)),
    ),
    out_shape=(
        jax.ShapeDtypeStruct((NF, 128), jnp.float32),
        jax.ShapeDtypeStruct((NF, 128), jnp.float32),
    ),
)


def _tc2_body(a0_ref, a1_ref, scaled1_ref, dinv_ref, b1_ref, w2blk_ref,
              out_ref):
    dinv = dinv_ref[...]
    h1 = jnp.maximum(
        dinv * (a0_ref[...] + a1_ref[...] + scaled1_ref[...]) + b1_ref[...],
        0.0)
    h2 = jnp.dot(h1, w2blk_ref[...], preferred_element_type=jnp.float32)
    out_ref[...] = h2 * dinv


_tc2 = pl.pallas_call(
    _tc2_body,
    grid=(GRID,),
    in_specs=[
        pl.BlockSpec((BLKF, 128), lambda g: (g, 0)),
        pl.BlockSpec((BLKF, 128), lambda g: (g + GRID, 0)),
        pl.BlockSpec((BLKF, 128), lambda g: (g, 0)),
        pl.BlockSpec((BLKF, 128), lambda g: (g, 0)),
        pl.BlockSpec((1, 128), lambda g: (0, 0)),
        pl.BlockSpec((128, 128), lambda g: (0, 0)),
    ],
    out_specs=pl.BlockSpec((BLKF, 128), lambda g: (g, 0)),
    out_shape=jax.ShapeDtypeStruct((NF, 128), jnp.float32),
)


def _tc3_body(a0_ref, a1_ref, scaled2_ref, dinv_ref, b2_ref, g8_ref, out_ref):
    logits = dinv_ref[...] * (
        a0_ref[...] + a1_ref[...] + scaled2_ref[...]
    ) + b2_ref[...]
    m = jnp.max(logits, axis=1, keepdims=True)
    e = jnp.exp(logits - m)
    s = jnp.dot(e, g8_ref[...], preferred_element_type=jnp.float32)
    out_ref[...] = logits - m - jnp.log(s)


_tc3 = pl.pallas_call(
    _tc3_body,
    grid=(GRID,),
    in_specs=[
        pl.BlockSpec((BLKF, 128), lambda g: (g, 0)),
        pl.BlockSpec((BLKF, 128), lambda g: (g + GRID, 0)),
        pl.BlockSpec((BLKF, 128), lambda g: (g, 0)),
        pl.BlockSpec((BLKF, 128), lambda g: (g, 0)),
        pl.BlockSpec((1, 128), lambda g: (0, 0)),
        pl.BlockSpec((128, 128), lambda g: (0, 0)),
    ],
    out_specs=pl.BlockSpec((BLKF, 128), lambda g: (g, 0)),
    out_shape=jax.ShapeDtypeStruct((NF, 128), jnp.float32),
)


def kernel(x, edge_index, W1, b1, W2, b2):
    srcp, dstp = _prep(edge_index.astype(jnp.int32))
    src2d = srcp.reshape(NCHUNKS, CHUNK)
    dst2d = dstp.reshape(NCHUNKS, CHUNK)

    w2p = jnp.zeros((HID, HID), jnp.float32).at[:, :C].set(W2)
    w2blk = jnp.kron(jnp.eye(PACK, dtype=jnp.float32), w2p)
    g8 = jnp.kron(jnp.eye(PACK, dtype=jnp.float32),
                  jnp.ones((HID, HID), jnp.float32))
    b1f = jnp.tile(b1.reshape(1, HID), (1, PACK))
    b2p = jnp.full((HID,), -1e30, jnp.float32).at[:C].set(b2)
    b2f = jnp.tile(b2p.reshape(1, HID), (1, PACK))

    w1s = jnp.kron(jnp.eye(PACK, dtype=jnp.float32), W1)

    degrep = _deg_kernel(dst2d)
    degf = degrep.reshape(NC * NF, 128)

    scaled1, dinv = _tc1(degf, degf, *([x] * PACK), w1s)

    agg1 = _prop_kernel(scaled1.reshape(NPAD, HID), src2d, dst2d)
    agg1f = agg1.reshape(NC * NF, 128)
    scaled2 = _tc2(agg1f, agg1f, scaled1, dinv, b1f, w2blk)

    agg2 = _prop_kernel(scaled2.reshape(NPAD, HID), src2d, dst2d)
    agg2f = agg2.reshape(NC * NF, 128)
    lp = _tc3(agg2f, agg2f, scaled2, dinv, b2f, g8)

    return lp.reshape(NF, PACK, HID).transpose(1, 0, 2).reshape(NPAD, HID)[:N, :C]


# speedup vs baseline: 1.1620x; 1.0061x over previous
import functools

import jax
import jax.numpy as jnp
from jax import lax
from jax.experimental import pallas as pl
from jax.experimental.pallas import tpu as pltpu
from jax.experimental.pallas import tpu_sc as plsc

N = 10000
E = 320000
F_IN = 128
HID = 16
C = 10

NC = 2
NS = 16
NW = NC * NS
CHUNK = 128
TPC = 80
EPAD = NW * TPC * CHUNK
ROWS_PER_TILE = 640
NPAD = NS * ROWS_PER_TILE
LAST = N - (NS - 1) * ROWS_PER_TILE
DEPTH = 8
LAG = 4
NCHUNKS = EPAD // CHUNK
TPC_F = 92
TPC_S = 68
DTPC_F = 144
DTPC_S = 16

PACK = 128 // HID
NF = NPAD // PACK
BLKF = 256
GRID = NF // BLKF
BLKN = BLKF * PACK

_mesh = plsc.VectorSubcoreMesh(core_axis_name="c", subcore_axis_name="s")
_sc_params = pltpu.CompilerParams(use_tc_tiling_on_sc=False)


T_PAD = PACK * (N % NF) + N // NF


def _slot(n):
    a = ((n >> 8) * 52429) >> 18
    r = n - a * NF
    return PACK * r + a


def _prep_body(e_ref, src_ref, dst_ref):
    src_ref[pl.ds(0, E)] = _slot(e_ref[0, :])
    src_ref[pl.ds(E, EPAD - E)] = jnp.full((EPAD - E,), T_PAD, jnp.int32)
    dst_ref[pl.ds(0, E)] = _slot(e_ref[1, :])
    dst_ref[pl.ds(E, EPAD - E)] = jnp.full((EPAD - E,), T_PAD, jnp.int32)


_prep = pl.pallas_call(
    _prep_body,
    out_shape=(
        jax.ShapeDtypeStruct((EPAD,), jnp.int32),
        jax.ShapeDtypeStruct((EPAD,), jnp.int32),
    ),
)


@functools.partial(
    pl.kernel,
    out_type=jax.ShapeDtypeStruct((NC * NPAD, HID), jnp.float32),
    mesh=_mesh,
    scratch_types=[
        pltpu.VMEM_SHARED((NPAD,), jnp.float32),
        pltpu.VMEM((DTPC_F, CHUNK), jnp.int32),
        pltpu.VMEM((CHUNK,), jnp.float32),
        pltpu.VMEM((ROWS_PER_TILE + 16,), jnp.float32),
        pltpu.VMEM((ROWS_PER_TILE, HID), jnp.float32),
        pltpu.SemaphoreType.DMA,
    ],
    compiler_params=_sc_params,
)
def _deg_kernel(dst_hbm, out_hbm, deg_sh, idx_v, ones_v, bounce_v, rep_v, sem):
    c = lax.axis_index("c")
    s = lax.axis_index("s")
    tpc_c = jnp.where(c == 0, DTPC_F, DTPC_S)

    for i in range(CHUNK // 16):
        ones_v[pl.ds(i * 16, 16)] = jnp.ones((16,), jnp.float32)

    def zero_body(i, _):
        bounce_v[pl.ds(i * 16, 16)] = jnp.zeros((16,), jnp.float32)
        return 0

    lax.fori_loop(0, ROWS_PER_TILE // 16 + 1, zero_body, 0)
    pltpu.sync_copy(bounce_v.at[pl.ds(0, ROWS_PER_TILE)],
                    deg_sh.at[pl.ds(s * ROWS_PER_TILE, ROWS_PER_TILE)])

    @pl.when(c == 0)
    def _():
        pltpu.sync_copy(dst_hbm.at[pl.ds(s * DTPC_F, DTPC_F)], idx_v)

    @pl.when(c == 1)
    def _():
        pltpu.sync_copy(dst_hbm.at[pl.ds(NS * DTPC_F + s * DTPC_S, DTPC_S)],
                        idx_v.at[pl.ds(0, DTPC_S)])

    plsc.subcore_barrier()

    def scat(j):
        return pltpu.make_async_copy(ones_v, deg_sh.at[idx_v.at[j]], sem)

    def body(j, _):
        scat(j).start(add=True)

        @pl.when(j >= DEPTH)
        def _():
            scat(0).wait()

        return 0

    lax.fori_loop(0, tpc_c, body, 0)
    for _ in range(DEPTH):
        scat(0).wait()
    plsc.subcore_barrier()

    pltpu.sync_copy(deg_sh.at[pl.ds(s * ROWS_PER_TILE, ROWS_PER_TILE)],
                    bounce_v.at[pl.ds(0, ROWS_PER_TILE)])

    def rep_body(i, _):
        v = bounce_v[pl.ds(i, 16)]
        rep_v[i, :] = jnp.full((HID,), v[0], jnp.float32)
        return 0

    lax.fori_loop(0, ROWS_PER_TILE, rep_body, 0)

    pltpu.sync_copy(
        rep_v, out_hbm.at[pl.ds(c * NPAD + s * ROWS_PER_TILE, ROWS_PER_TILE)]
    )


@functools.partial(
    pl.kernel,
    out_type=jax.ShapeDtypeStruct((NC * NPAD, HID), jnp.float32),
    mesh=_mesh,
    scratch_types=[
        pltpu.VMEM_SHARED((NPAD, HID), jnp.float32),
        pltpu.VMEM_SHARED((NPAD, HID), jnp.float32),
        pltpu.VMEM((TPC_F, CHUNK), jnp.int32),
        pltpu.VMEM((TPC_F, CHUNK), jnp.int32),
        pltpu.VMEM((DEPTH, CHUNK, HID), jnp.float32),
        pltpu.VMEM((ROWS_PER_TILE, HID), jnp.float32),
        pltpu.SemaphoreType.DMA,
        pltpu.SemaphoreType.DMA,
    ],
    compiler_params=_sc_params,
)
def _prop_kernel(table_hbm, src_hbm, dst_hbm, out_hbm,
                 agg_sh, table_sh, isrc_v, idst_v, rows_v, bounce_v,
                 sem_g, sem_s):
    c = lax.axis_index("c")
    s = lax.axis_index("s")
    tpc_c = jnp.where(c == 0, TPC_F, TPC_S)

    pltpu.sync_copy(
        table_hbm.at[pl.ds(s * ROWS_PER_TILE, ROWS_PER_TILE)],
        table_sh.at[pl.ds(s * ROWS_PER_TILE, ROWS_PER_TILE)])

    def zero_body(i, _):
        bounce_v[i, :] = jnp.zeros((HID,), jnp.float32)
        return 0

    lax.fori_loop(0, ROWS_PER_TILE, zero_body, 0)
    pltpu.sync_copy(bounce_v, agg_sh.at[pl.ds(s * ROWS_PER_TILE, ROWS_PER_TILE)])

    @pl.when(c == 0)
    def _():
        pltpu.sync_copy(src_hbm.at[pl.ds(s * TPC_F, TPC_F)], isrc_v)
        pltpu.sync_copy(dst_hbm.at[pl.ds(s * TPC_F, TPC_F)], idst_v)

    @pl.when(c == 1)
    def _():
        base = NS * TPC_F + s * TPC_S
        pltpu.sync_copy(src_hbm.at[pl.ds(base, TPC_S)],
                        isrc_v.at[pl.ds(0, TPC_S)])
        pltpu.sync_copy(dst_hbm.at[pl.ds(base, TPC_S)],
                        idst_v.at[pl.ds(0, TPC_S)])

    plsc.subcore_barrier()

    def gath(j, b):
        return pltpu.make_async_copy(table_sh.at[isrc_v.at[j]], rows_v.at[b],
                                     sem_g)

    def scat(j, b):
        return pltpu.make_async_copy(rows_v.at[b], agg_sh.at[idst_v.at[j]],
                                     sem_s)

    def body(j, _):
        b = lax.rem(j, DEPTH)

        @pl.when(j >= DEPTH)
        def _():
            scat(0, 0).wait()

        gath(j, b).start()

        @pl.when(j >= LAG)
        def _():
            gath(0, 0).wait()
            jj = j - LAG
            scat(jj, lax.rem(jj, DEPTH)).start(add=True)

        return 0

    lax.fori_loop(0, tpc_c, body, 0)

    def tail(j, _):
        gath(0, 0).wait()
        jj = j - LAG
        scat(jj, lax.rem(jj, DEPTH)).start(add=True)
        return 0

    lax.fori_loop(tpc_c, tpc_c + LAG, tail, 0)
    for _ in range(DEPTH):
        scat(0, 0).wait()
    plsc.subcore_barrier()

    pltpu.sync_copy(agg_sh.at[pl.ds(s * ROWS_PER_TILE, ROWS_PER_TILE)], bounce_v)

    pltpu.sync_copy(
        bounce_v, out_hbm.at[pl.ds(c * NPAD + s * ROWS_PER_TILE, ROWS_PER_TILE)]
    )


def _tc1_body(d0_ref, d1_ref, *rest):
    xrefs = rest[:PACK]
    w1s_ref, scaled_ref, dinv_ref = rest[PACK], rest[PACK + 1], rest[PACK + 2]
    dinv = lax.rsqrt(d0_ref[...] + d1_ref[...] + 1.0)
    xcat = jnp.concatenate([xr[...] for xr in xrefs], axis=1)
    ri = lax.broadcasted_iota(jnp.int32, (BLKF, PACK * F_IN), 0) \
        + pl.program_id(0) * BLKF
    ci = lax.broadcasted_iota(jnp.int32, (BLKF, PACK * F_IN), 1)
    pad = (ci >= (PACK - 1) * F_IN) & (ri >= (N % NF))
    xcat = jnp.where(pad, 0.0, xcat)
    hf = jnp.dot(xcat, w1s_ref[...], preferred_element_type=jnp.float32)
    scaled_ref[...] = hf * dinv
    dinv_ref[...] = dinv


def _xa_spec(a):
    return pl.BlockSpec((BLKF, F_IN), lambda g, a=a: (g + a * (NF // BLKF), 0))


_tc1 = pl.pallas_call(
    _tc1_body,
    grid=(GRID,),
    in_specs=[
        pl.BlockSpec((BLKF, 128), lambda g: (g, 0)),
        pl.BlockSpec((BLKF, 128), lambda g: (g + GRID, 0)),
    ] + [_xa_spec(a) for a in range(PACK)] + [
        pl.BlockSpec((PACK * F_IN, 128), lambda g: (0, 0)),
    ],
    out_specs=(
        pl.BlockSpec((BLKF, 128), lambda g: (g, 0)),
        pl.BlockSpec((BLKF, 128), lambda g: (g, 0)),
    ),
    out_shape=(
        jax.ShapeDtypeStruct((NF, 128), jnp.float32),
        jax.ShapeDtypeStruct((NF, 128), jnp.float32),
    ),
)


def _tc2_body(a0_ref, a1_ref, scaled1_ref, dinv_ref, b1_ref, w2blk_ref,
              out_ref):
    dinv = dinv_ref[...]
    h1 = jnp.maximum(
        dinv * (a0_ref[...] + a1_ref[...] + scaled1_ref[...]) + b1_ref[...],
        0.0)
    h2 = jnp.dot(h1, w2blk_ref[...], preferred_element_type=jnp.float32)
    out_ref[...] = h2 * dinv


_tc2 = pl.pallas_call(
    _tc2_body,
    grid=(GRID,),
    in_specs=[
        pl.BlockSpec((BLKF, 128), lambda g: (g, 0)),
        pl.BlockSpec((BLKF, 128), lambda g: (g + GRID, 0)),
        pl.BlockSpec((BLKF, 128), lambda g: (g, 0)),
        pl.BlockSpec((BLKF, 128), lambda g: (g, 0)),
        pl.BlockSpec((1, 128), lambda g: (0, 0)),
        pl.BlockSpec((128, 128), lambda g: (0, 0)),
    ],
    out_specs=pl.BlockSpec((BLKF, 128), lambda g: (g, 0)),
    out_shape=jax.ShapeDtypeStruct((NF, 128), jnp.float32),
)


def _tc3_body(a0_ref, a1_ref, scaled2_ref, dinv_ref, b2_ref, g8_ref, out_ref):
    logits = dinv_ref[...] * (
        a0_ref[...] + a1_ref[...] + scaled2_ref[...]
    ) + b2_ref[...]
    m = jnp.max(logits, axis=1, keepdims=True)
    e = jnp.exp(logits - m)
    s = jnp.dot(e, g8_ref[...], preferred_element_type=jnp.float32)
    out_ref[...] = logits - m - jnp.log(s)


_tc3 = pl.pallas_call(
    _tc3_body,
    grid=(GRID,),
    in_specs=[
        pl.BlockSpec((BLKF, 128), lambda g: (g, 0)),
        pl.BlockSpec((BLKF, 128), lambda g: (g + GRID, 0)),
        pl.BlockSpec((BLKF, 128), lambda g: (g, 0)),
        pl.BlockSpec((BLKF, 128), lambda g: (g, 0)),
        pl.BlockSpec((1, 128), lambda g: (0, 0)),
        pl.BlockSpec((128, 128), lambda g: (0, 0)),
    ],
    out_specs=pl.BlockSpec((BLKF, 128), lambda g: (g, 0)),
    out_shape=jax.ShapeDtypeStruct((NF, 128), jnp.float32),
)


def kernel(x, edge_index, W1, b1, W2, b2):
    srcp, dstp = _prep(edge_index.astype(jnp.int32))
    src2d = srcp.reshape(NCHUNKS, CHUNK)
    dst2d = dstp.reshape(NCHUNKS, CHUNK)

    w2p = jnp.zeros((HID, HID), jnp.float32).at[:, :C].set(W2)
    w2blk = jnp.kron(jnp.eye(PACK, dtype=jnp.float32), w2p)
    g8 = jnp.kron(jnp.eye(PACK, dtype=jnp.float32),
                  jnp.ones((HID, HID), jnp.float32))
    b1f = jnp.tile(b1.reshape(1, HID), (1, PACK))
    b2p = jnp.full((HID,), -1e30, jnp.float32).at[:C].set(b2)
    b2f = jnp.tile(b2p.reshape(1, HID), (1, PACK))

    w1s = jnp.kron(jnp.eye(PACK, dtype=jnp.float32), W1)

    degrep = _deg_kernel(dst2d)
    degf = degrep.reshape(NC * NF, 128)

    scaled1, dinv = _tc1(degf, degf, *([x] * PACK), w1s)

    agg1 = _prop_kernel(scaled1.reshape(NPAD, HID), src2d, dst2d)
    agg1f = agg1.reshape(NC * NF, 128)
    scaled2 = _tc2(agg1f, agg1f, scaled1, dinv, b1f, w2blk)

    agg2 = _prop_kernel(scaled2.reshape(NPAD, HID), src2d, dst2d)
    agg2f = agg2.reshape(NC * NF, 128)
    lp = _tc3(agg2f, agg2f, scaled2, dinv, b2f, g8)

    return lp.reshape(NF, PACK, HID).transpose(1, 0, 2).reshape(NPAD, HID)[:N, :C]


# speedup vs baseline: 1.1758x; 1.0119x over previous
import functools

import jax
import jax.numpy as jnp
from jax import lax
from jax.experimental import pallas as pl
from jax.experimental.pallas import tpu as pltpu
from jax.experimental.pallas import tpu_sc as plsc

N = 10000
E = 320000
F_IN = 128
HID = 16
C = 10

NC = 2
NS = 16
NW = NC * NS
CHUNK = 128
TPC = 80
EPAD = NW * TPC * CHUNK
ROWS_PER_TILE = 640
NPAD = NS * ROWS_PER_TILE
LAST = N - (NS - 1) * ROWS_PER_TILE
DEPTH = 8
LAG = 4
NCHUNKS = EPAD // CHUNK
TPC_F = 96
TPC_S = 64
DTPC_F = 144
DTPC_S = 16

PACK = 128 // HID
NF = NPAD // PACK
BLKF = 256
GRID = NF // BLKF
BLKN = BLKF * PACK

_mesh = plsc.VectorSubcoreMesh(core_axis_name="c", subcore_axis_name="s")
_sc_params = pltpu.CompilerParams(use_tc_tiling_on_sc=False)


T_PAD = PACK * (N % NF) + N // NF


def _slot(n):
    a = ((n >> 8) * 52429) >> 18
    r = n - a * NF
    return PACK * r + a


def _prep_body(e_ref, src_ref, dst_ref):
    src_ref[pl.ds(0, E)] = _slot(e_ref[0, :])
    src_ref[pl.ds(E, EPAD - E)] = jnp.full((EPAD - E,), T_PAD, jnp.int32)
    dst_ref[pl.ds(0, E)] = _slot(e_ref[1, :])
    dst_ref[pl.ds(E, EPAD - E)] = jnp.full((EPAD - E,), T_PAD, jnp.int32)


_prep = pl.pallas_call(
    _prep_body,
    out_shape=(
        jax.ShapeDtypeStruct((EPAD,), jnp.int32),
        jax.ShapeDtypeStruct((EPAD,), jnp.int32),
    ),
)


@functools.partial(
    pl.kernel,
    out_type=jax.ShapeDtypeStruct((NC * NPAD, HID), jnp.float32),
    mesh=_mesh,
    scratch_types=[
        pltpu.VMEM_SHARED((NPAD,), jnp.float32),
        pltpu.VMEM((DTPC_F, CHUNK), jnp.int32),
        pltpu.VMEM((CHUNK,), jnp.float32),
        pltpu.VMEM((ROWS_PER_TILE + 16,), jnp.float32),
        pltpu.VMEM((ROWS_PER_TILE, HID), jnp.float32),
        pltpu.SemaphoreType.DMA,
    ],
    compiler_params=_sc_params,
)
def _deg_kernel(dst_hbm, out_hbm, deg_sh, idx_v, ones_v, bounce_v, rep_v, sem):
    c = lax.axis_index("c")
    s = lax.axis_index("s")
    tpc_c = jnp.where(c == 0, DTPC_F, DTPC_S)

    for i in range(CHUNK // 16):
        ones_v[pl.ds(i * 16, 16)] = jnp.ones((16,), jnp.float32)

    def zero_body(i, _):
        bounce_v[pl.ds(i * 16, 16)] = jnp.zeros((16,), jnp.float32)
        return 0

    lax.fori_loop(0, ROWS_PER_TILE // 16 + 1, zero_body, 0)
    pltpu.sync_copy(bounce_v.at[pl.ds(0, ROWS_PER_TILE)],
                    deg_sh.at[pl.ds(s * ROWS_PER_TILE, ROWS_PER_TILE)])

    @pl.when(c == 0)
    def _():
        pltpu.sync_copy(dst_hbm.at[pl.ds(s * DTPC_F, DTPC_F)], idx_v)

    @pl.when(c == 1)
    def _():
        pltpu.sync_copy(dst_hbm.at[pl.ds(NS * DTPC_F + s * DTPC_S, DTPC_S)],
                        idx_v.at[pl.ds(0, DTPC_S)])

    plsc.subcore_barrier()

    def scat(j):
        return pltpu.make_async_copy(ones_v, deg_sh.at[idx_v.at[j]], sem)

    def body(j, _):
        scat(j).start(add=True)

        @pl.when(j >= DEPTH)
        def _():
            scat(0).wait()

        return 0

    lax.fori_loop(0, tpc_c, body, 0)
    for _ in range(DEPTH):
        scat(0).wait()
    plsc.subcore_barrier()

    pltpu.sync_copy(deg_sh.at[pl.ds(s * ROWS_PER_TILE, ROWS_PER_TILE)],
                    bounce_v.at[pl.ds(0, ROWS_PER_TILE)])

    def rep_body(i, _):
        v = bounce_v[pl.ds(i, 16)]
        rep_v[i, :] = jnp.full((HID,), v[0], jnp.float32)
        return 0

    lax.fori_loop(0, ROWS_PER_TILE, rep_body, 0)

    pltpu.sync_copy(
        rep_v, out_hbm.at[pl.ds(c * NPAD + s * ROWS_PER_TILE, ROWS_PER_TILE)]
    )


@functools.partial(
    pl.kernel,
    out_type=jax.ShapeDtypeStruct((NC * NPAD, HID), jnp.float32),
    mesh=_mesh,
    scratch_types=[
        pltpu.VMEM_SHARED((NPAD, HID), jnp.float32),
        pltpu.VMEM_SHARED((NPAD, HID), jnp.float32),
        pltpu.VMEM((TPC_F, CHUNK), jnp.int32),
        pltpu.VMEM((TPC_F, CHUNK), jnp.int32),
        pltpu.VMEM((DEPTH, CHUNK, HID), jnp.float32),
        pltpu.VMEM((ROWS_PER_TILE, HID), jnp.float32),
        pltpu.SemaphoreType.DMA,
        pltpu.SemaphoreType.DMA,
    ],
    compiler_params=_sc_params,
)
def _prop_kernel(table_hbm, src_hbm, dst_hbm, out_hbm,
                 agg_sh, table_sh, isrc_v, idst_v, rows_v, bounce_v,
                 sem_g, sem_s):
    c = lax.axis_index("c")
    s = lax.axis_index("s")
    tpc_c = jnp.where(c == 0, TPC_F, TPC_S)

    pltpu.sync_copy(
        table_hbm.at[pl.ds(s * ROWS_PER_TILE, ROWS_PER_TILE)],
        table_sh.at[pl.ds(s * ROWS_PER_TILE, ROWS_PER_TILE)])

    def zero_body(i, _):
        bounce_v[i, :] = jnp.zeros((HID,), jnp.float32)
        return 0

    lax.fori_loop(0, ROWS_PER_TILE, zero_body, 0)
    pltpu.sync_copy(bounce_v, agg_sh.at[pl.ds(s * ROWS_PER_TILE, ROWS_PER_TILE)])

    @pl.when(c == 0)
    def _():
        pltpu.sync_copy(src_hbm.at[pl.ds(s * TPC_F, TPC_F)], isrc_v)
        pltpu.sync_copy(dst_hbm.at[pl.ds(s * TPC_F, TPC_F)], idst_v)

    @pl.when(c == 1)
    def _():
        base = NS * TPC_F + s * TPC_S
        pltpu.sync_copy(src_hbm.at[pl.ds(base, TPC_S)],
                        isrc_v.at[pl.ds(0, TPC_S)])
        pltpu.sync_copy(dst_hbm.at[pl.ds(base, TPC_S)],
                        idst_v.at[pl.ds(0, TPC_S)])

    plsc.subcore_barrier()

    def gath(j, b):
        return pltpu.make_async_copy(table_sh.at[isrc_v.at[j]], rows_v.at[b],
                                     sem_g)

    def scat(j, b):
        return pltpu.make_async_copy(rows_v.at[b], agg_sh.at[idst_v.at[j]],
                                     sem_s)

    def body(j, _):
        b = lax.rem(j, DEPTH)

        @pl.when(j >= DEPTH)
        def _():
            scat(0, 0).wait()

        gath(j, b).start()

        @pl.when(j >= LAG)
        def _():
            gath(0, 0).wait()
            jj = j - LAG
            scat(jj, lax.rem(jj, DEPTH)).start(add=True)

        return 0

    lax.fori_loop(0, tpc_c, body, 0)

    def tail(j, _):
        gath(0, 0).wait()
        jj = j - LAG
        scat(jj, lax.rem(jj, DEPTH)).start(add=True)
        return 0

    lax.fori_loop(tpc_c, tpc_c + LAG, tail, 0)
    for _ in range(DEPTH):
        scat(0, 0).wait()
    plsc.subcore_barrier()

    pltpu.sync_copy(agg_sh.at[pl.ds(s * ROWS_PER_TILE, ROWS_PER_TILE)], bounce_v)

    pltpu.sync_copy(
        bounce_v, out_hbm.at[pl.ds(c * NPAD + s * ROWS_PER_TILE, ROWS_PER_TILE)]
    )


def _tc1_body(d0_ref, d1_ref, *rest):
    xrefs = rest[:PACK]
    w1s_ref, scaled_ref, dinv_ref = rest[PACK], rest[PACK + 1], rest[PACK + 2]
    dinv = lax.rsqrt(d0_ref[...] + d1_ref[...] + 1.0)
    xcat = jnp.concatenate([xr[...] for xr in xrefs], axis=1)
    ri = lax.broadcasted_iota(jnp.int32, (BLKF, PACK * F_IN), 0) \
        + pl.program_id(0) * BLKF
    ci = lax.broadcasted_iota(jnp.int32, (BLKF, PACK * F_IN), 1)
    pad = (ci >= (PACK - 1) * F_IN) & (ri >= (N % NF))
    xcat = jnp.where(pad, 0.0, xcat)
    hf = jnp.dot(xcat, w1s_ref[...], preferred_element_type=jnp.float32)
    scaled_ref[...] = hf * dinv
    dinv_ref[...] = dinv


def _xa_spec(a):
    return pl.BlockSpec((BLKF, F_IN), lambda g, a=a: (g + a * (NF // BLKF), 0))


_tc1 = pl.pallas_call(
    _tc1_body,
    grid=(GRID,),
    in_specs=[
        pl.BlockSpec((BLKF, 128), lambda g: (g, 0)),
        pl.BlockSpec((BLKF, 128), lambda g: (g + GRID, 0)),
    ] + [_xa_spec(a) for a in range(PACK)] + [
        pl.BlockSpec((PACK * F_IN, 128), lambda g: (0, 0)),
    ],
    out_specs=(
        pl.BlockSpec((BLKF, 128), lambda g: (g, 0)),
        pl.BlockSpec((BLKF, 128), lambda g: (g, 0)),
    ),
    out_shape=(
        jax.ShapeDtypeStruct((NF, 128), jnp.float32),
        jax.ShapeDtypeStruct((NF, 128), jnp.float32),
    ),
)


def _tc2_body(a0_ref, a1_ref, scaled1_ref, dinv_ref, b1_ref, w2blk_ref,
              out_ref):
    dinv = dinv_ref[...]
    h1 = jnp.maximum(
        dinv * (a0_ref[...] + a1_ref[...] + scaled1_ref[...]) + b1_ref[...],
        0.0)
    h2 = jnp.dot(h1, w2blk_ref[...], preferred_element_type=jnp.float32)
    out_ref[...] = h2 * dinv


_tc2 = pl.pallas_call(
    _tc2_body,
    grid=(GRID,),
    in_specs=[
        pl.BlockSpec((BLKF, 128), lambda g: (g, 0)),
        pl.BlockSpec((BLKF, 128), lambda g: (g + GRID, 0)),
        pl.BlockSpec((BLKF, 128), lambda g: (g, 0)),
        pl.BlockSpec((BLKF, 128), lambda g: (g, 0)),
        pl.BlockSpec((1, 128), lambda g: (0, 0)),
        pl.BlockSpec((128, 128), lambda g: (0, 0)),
    ],
    out_specs=pl.BlockSpec((BLKF, 128), lambda g: (g, 0)),
    out_shape=jax.ShapeDtypeStruct((NF, 128), jnp.float32),
)


def _tc3_body(a0_ref, a1_ref, scaled2_ref, dinv_ref, b2_ref, g8_ref, out_ref):
    logits = dinv_ref[...] * (
        a0_ref[...] + a1_ref[...] + scaled2_ref[...]
    ) + b2_ref[...]
    m = jnp.max(logits, axis=1, keepdims=True)
    e = jnp.exp(logits - m)
    s = jnp.dot(e, g8_ref[...], preferred_element_type=jnp.float32)
    out_ref[...] = logits - m - jnp.log(s)


_tc3 = pl.pallas_call(
    _tc3_body,
    grid=(GRID,),
    in_specs=[
        pl.BlockSpec((BLKF, 128), lambda g: (g, 0)),
        pl.BlockSpec((BLKF, 128), lambda g: (g + GRID, 0)),
        pl.BlockSpec((BLKF, 128), lambda g: (g, 0)),
        pl.BlockSpec((BLKF, 128), lambda g: (g, 0)),
        pl.BlockSpec((1, 128), lambda g: (0, 0)),
        pl.BlockSpec((128, 128), lambda g: (0, 0)),
    ],
    out_specs=pl.BlockSpec((BLKF, 128), lambda g: (g, 0)),
    out_shape=jax.ShapeDtypeStruct((NF, 128), jnp.float32),
)


def kernel(x, edge_index, W1, b1, W2, b2):
    srcp, dstp = _prep(edge_index.astype(jnp.int32))
    src2d = srcp.reshape(NCHUNKS, CHUNK)
    dst2d = dstp.reshape(NCHUNKS, CHUNK)

    w2p = jnp.zeros((HID, HID), jnp.float32).at[:, :C].set(W2)
    w2blk = jnp.kron(jnp.eye(PACK, dtype=jnp.float32), w2p)
    g8 = jnp.kron(jnp.eye(PACK, dtype=jnp.float32),
                  jnp.ones((HID, HID), jnp.float32))
    b1f = jnp.tile(b1.reshape(1, HID), (1, PACK))
    b2p = jnp.full((HID,), -1e30, jnp.float32).at[:C].set(b2)
    b2f = jnp.tile(b2p.reshape(1, HID), (1, PACK))

    w1s = jnp.kron(jnp.eye(PACK, dtype=jnp.float32), W1)

    degrep = _deg_kernel(dst2d)
    degf = degrep.reshape(NC * NF, 128)

    scaled1, dinv = _tc1(degf, degf, *([x] * PACK), w1s)

    agg1 = _prop_kernel(scaled1.reshape(NPAD, HID), src2d, dst2d)
    agg1f = agg1.reshape(NC * NF, 128)
    scaled2 = _tc2(agg1f, agg1f, scaled1, dinv, b1f, w2blk)

    agg2 = _prop_kernel(scaled2.reshape(NPAD, HID), src2d, dst2d)
    agg2f = agg2.reshape(NC * NF, 128)
    lp = _tc3(agg2f, agg2f, scaled2, dinv, b2f, g8)

    return lp.reshape(NF, PACK, HID).transpose(1, 0, 2).reshape(NPAD, HID)[:N, :C]
